# baseline (speedup 1.0000x reference)
import functools

import jax
import jax.numpy as jnp
from jax import lax
from jax.experimental import pallas as pl
from jax.experimental.pallas import tpu as pltpu
from jax.experimental.pallas import tpu_sc as plsc

_EPS = 1e-07
_B, _H, _W = 16, 384, 384
_HALF_ROWS = _H // 2
_CHUNK_ROWS = 16
_N_CHUNKS = _HALF_ROWS // _CHUNK_ROWS
_LANES = 16
_VPR = _W // _LANES


def _sc_body(rt, af, rp, ap, cf, fg, bg, out,
             b0_rt, b0_af, b0_rp, b0_ap, b0_cf, b0_fg, b0_bg,
             b1_rt, b1_af, b1_rp, b1_ap, b1_cf, b1_fg, b1_bg,
             b_out, sem0, sem1):
    sample = lax.axis_index("s")
    half = lax.axis_index("c")
    row_base = half * _HALF_ROWS

    slots = ((b0_rt, b0_af, b0_rp, b0_ap, b0_cf, b0_fg, b0_bg, sem0),
             (b1_rt, b1_af, b1_rp, b1_ap, b1_cf, b1_fg, b1_bg, sem1))
    hbms = (rt, af, rp, ap, cf, fg, bg)

    def issue(chunk, slot):
        w0 = (row_base + chunk * _CHUNK_ROWS) * _W
        sem = slot[7]
        for h, b in zip(hbms, slot[:7]):
            pltpu.async_copy(h.at[sample, pl.ds(w0, _CHUNK_ROWS * _W)], b, sem)

    def drain(slot):
        sem = slot[7]
        for h, b in zip(hbms, slot[:7]):
            pltpu.make_async_copy(
                h.at[sample, pl.ds(0, _CHUNK_ROWS * _W)], b, sem).wait()

    def compute(slot, carry):
        b_rt, b_af, b_rp, b_ap, b_cf, b_fg, b_bg = slot[:7]

        def row_step(r, c2):
            M, SL, SC, SLFG, SCFG = c2
            for j in range(_VPR):
                sl = pl.ds(r * _W + j * _LANES, _LANES)
                PROBE = True
                if PROBE:
                    if j == 0:
                        M = M + b_cf[sl]
                    continue
                vcf = b_cf[sl]
                conf = jnp.where(vcf >= 0.5, vcf, 0.0)
                l = (jnp.abs(b_rt[sl] - b_rp[sl])
                     + jnp.abs(b_af[sl] - b_ap[sl])) * conf
                vfg = b_fg[sl]
                vbg = b_bg[sl]
                nl = l * vbg
                tie = nl >= M
                rst = nl > M
                M = jnp.maximum(M, nl)
                SLFG = SLFG + l * vfg
                SCFG = SCFG + conf * vfg
                SL = jnp.where(rst, 0.0, SL) + jnp.where(tie, nl, 0.0)
                SC = (jnp.where(rst, 0.0, SC)
                      + jnp.where(tie, conf * vbg, 0.0))
            return (M, SL, SC, SLFG, SCFG)

        return lax.fori_loop(0, _CHUNK_ROWS, row_step, carry)

    issue(0, slots[0])
    issue(1, slots[1])

    def pair_step(g, carry):
        for p in range(2):
            slot = slots[p]
            drain(slot)
            carry = compute(slot, carry)

            @pl.when(g < _N_CHUNKS // 2 - 1)
            def _():
                issue(2 * g + 2 + p, slot)
        return carry

    z = jnp.zeros((_LANES,), jnp.float32)
    M, SL, SC, SLFG, SCFG = lax.fori_loop(
        0, _N_CHUNKS // 2, pair_step, (z, z, z, z, z))

    b_out[0, :] = M
    b_out[1, :] = SL
    b_out[2, :] = SC
    b_out[3, :] = SLFG
    b_out[4, :] = SCFG
    pltpu.sync_copy(b_out, out.at[half * _B + sample])


@functools.partial(
    pl.kernel,
    out_type=jax.ShapeDtypeStruct((32, 5, _LANES), jnp.float32),
    mesh=plsc.VectorSubcoreMesh(core_axis_name="c", subcore_axis_name="s"),
    scratch_types=(
        [pltpu.VMEM((_CHUNK_ROWS * _W,), jnp.float32)] * 14
        + [pltpu.VMEM((5, _LANES), jnp.float32)]
        + [pltpu.SemaphoreType.DMA, pltpu.SemaphoreType.DMA]
    ),
)
def _sc_partials(*args):
    _sc_body(*args)


def _combine_body(p_ref, o_ref):
    p = p_ref[...]
    a = p[:_B]
    b = p[_B:]
    m = jnp.max(jnp.maximum(a[:, 0, :], b[:, 0, :]), axis=1, keepdims=True)
    wa = a[:, 0, :] >= m
    wb = b[:, 0, :] >= m
    sl = (jnp.sum(jnp.where(wa, a[:, 1, :], 0.0))
          + jnp.sum(jnp.where(wb, b[:, 1, :], 0.0)))
    sc = (jnp.sum(jnp.where(wa, a[:, 2, :], 0.0))
          + jnp.sum(jnp.where(wb, b[:, 2, :], 0.0)))
    num = sl + jnp.sum(a[:, 3, :]) + jnp.sum(b[:, 3, :])
    den = sc + jnp.sum(a[:, 4, :]) + jnp.sum(b[:, 4, :])
    o_ref[...] = num / (den + _EPS)


def kernel(region_true, affinity_true, region_pred, affinity_pred,
           confidence, fg_mask, bg_mask):
    flat = [x.reshape(_B, _H * _W) for x in
            (region_true, affinity_true, region_pred, affinity_pred,
             confidence, fg_mask, bg_mask)]
    partials = _sc_partials(*flat)
    out = pl.pallas_call(
        _combine_body,
        out_shape=jax.ShapeDtypeStruct((), jnp.float32),
        out_specs=pl.BlockSpec(memory_space=pltpu.SMEM),
    )(partials)
    return out


# speedup vs baseline: 3.0436x; 3.0436x over previous
import functools

import jax
import jax.numpy as jnp
from jax import lax
from jax.experimental import pallas as pl
from jax.experimental.pallas import tpu as pltpu
from jax.experimental.pallas import tpu_sc as plsc

_EPS = 1e-07
_B, _H, _W = 16, 384, 384
_SC_SAMPLES = 8
_SUB_PER_SAMPLE = 4
_SPAN = _H // _SUB_PER_SAMPLE
_CHUNK_ROWS = 16
_N_CHUNKS = _SPAN // _CHUNK_ROWS
_LANES = 16
_VPR = _W // _LANES


def _sc_body(rt, af, rp, ap, cf, fg, bg, out,
             b0_rt, b0_af, b0_rp, b0_ap, b0_cf, b0_fg, b0_bg,
             b1_rt, b1_af, b1_rp, b1_ap, b1_cf, b1_fg, b1_bg,
             b_out, sem0, sem1):
    wid = lax.axis_index("s") * 2 + lax.axis_index("c")
    sample = wid // _SUB_PER_SAMPLE
    quarter = wid - sample * _SUB_PER_SAMPLE
    row_base = quarter * _SPAN

    slots = ((b0_rt, b0_af, b0_rp, b0_ap, b0_cf, b0_fg, b0_bg, sem0),
             (b1_rt, b1_af, b1_rp, b1_ap, b1_cf, b1_fg, b1_bg, sem1))
    hbms = (rt, af, rp, ap, cf, fg, bg)

    def issue(chunk, slot):
        r0 = row_base + chunk * _CHUNK_ROWS
        sem = slot[7]
        for h, b in zip(hbms, slot[:7]):
            pltpu.async_copy(h.at[sample, pl.ds(r0, _CHUNK_ROWS), :], b, sem)

    def drain(slot):
        sem = slot[7]
        for h, b in zip(hbms, slot[:7]):
            pltpu.make_async_copy(
                h.at[sample, pl.ds(row_base, _CHUNK_ROWS), :], b, sem).wait()

    def compute(slot, carry):
        b_rt, b_af, b_rp, b_ap, b_cf, b_fg, b_bg = slot[:7]

        def row_step(r, c2):
            M, SL, SC, SLFG, SCFG = c2
            for j in range(_VPR):
                sl = pl.ds(j * _LANES, _LANES)
                vcf = b_cf[r, sl]
                conf = jnp.where(vcf >= 0.5, vcf, 0.0)
                l = (jnp.abs(b_rt[r, sl] - b_rp[r, sl])
                     + jnp.abs(b_af[r, sl] - b_ap[r, sl])) * conf
                vfg = b_fg[r, sl]
                vbg = b_bg[r, sl]
                nl = l * vbg
                tie = nl >= M
                rst = nl > M
                M = jnp.maximum(M, nl)
                SLFG = SLFG + l * vfg
                SCFG = SCFG + conf * vfg
                SL = jnp.where(rst, 0.0, SL) + jnp.where(tie, nl, 0.0)
                SC = (jnp.where(rst, 0.0, SC)
                      + jnp.where(tie, conf * vbg, 0.0))
            return (M, SL, SC, SLFG, SCFG)

        return lax.fori_loop(0, _CHUNK_ROWS, row_step, carry)

    issue(0, slots[0])
    issue(1, slots[1])

    def pair_step(g, carry):
        for p in range(2):
            slot = slots[p]
            drain(slot)
            carry = compute(slot, carry)

            @pl.when(g < _N_CHUNKS // 2 - 1)
            def _():
                issue(2 * g + 2 + p, slot)
        return carry

    z = jnp.zeros((_LANES,), jnp.float32)
    M, SL, SC, SLFG, SCFG = lax.fori_loop(
        0, _N_CHUNKS // 2, pair_step, (z, z, z, z, z))

    b_out[0, :] = M
    b_out[1, :] = SL
    b_out[2, :] = SC
    b_out[3, :] = SLFG
    b_out[4, :] = SCFG
    pltpu.sync_copy(b_out, out.at[wid])


@functools.partial(
    pl.kernel,
    out_type=jax.ShapeDtypeStruct((32, 5, _LANES), jnp.float32),
    mesh=plsc.VectorSubcoreMesh(core_axis_name="c", subcore_axis_name="s"),
    scratch_types=(
        [pltpu.VMEM((_CHUNK_ROWS, _W), jnp.float32)] * 14
        + [pltpu.VMEM((5, _LANES), jnp.float32)]
        + [pltpu.SemaphoreType.DMA, pltpu.SemaphoreType.DMA]
    ),
)
def _sc_partials(*args):
    _sc_body(*args)


def _tc_part_body(rt, af, rp, ap, cf, fg, bg, o_ref):
    vcf = cf[...]
    conf = jnp.where(vcf >= 0.5, vcf, 0.0)
    l = (jnp.abs(rt[...] - rp[...]) + jnp.abs(af[...] - ap[...])) * conf
    vfg = fg[...]
    vbg = bg[...]
    nl = l * vbg
    m = jnp.max(nl)
    hard = (vbg > 0.0) & (nl >= m)
    num = jnp.sum(l * vfg) + jnp.sum(jnp.where(hard, l, 0.0))
    den = jnp.sum(conf * vfg) + jnp.sum(jnp.where(hard, conf, 0.0))
    col = lax.broadcasted_iota(jnp.int32, (1, 1, 128), 2)
    o_ref[...] = jnp.where(col == 0, num, jnp.where(col == 1, den, 0.0))


def _tc_partials(*arrays):
    spec = pl.BlockSpec((1, _H, _W), lambda i: (i + _SC_SAMPLES, 0, 0))
    return pl.pallas_call(
        _tc_part_body,
        grid=(_B - _SC_SAMPLES,),
        in_specs=[spec] * 7,
        out_specs=pl.BlockSpec((1, 1, 128), lambda i: (i, 0, 0)),
        out_shape=jax.ShapeDtypeStruct(
            (_B - _SC_SAMPLES, 1, 128), jnp.float32),
    )(*arrays)


def _combine_body(p_ref, t_ref, o_ref):
    p = p_ref[...].reshape(_SC_SAMPLES, _SUB_PER_SAMPLE, 5, _LANES)
    M = p[:, :, 0, :]
    m = jnp.max(M.reshape(_SC_SAMPLES, -1), axis=1)[:, None, None]
    w = M >= m
    sl = jnp.sum(jnp.where(w, p[:, :, 1, :], 0.0))
    sc = jnp.sum(jnp.where(w, p[:, :, 2, :], 0.0))
    num = sl + jnp.sum(p[:, :, 3, :]) + jnp.sum(t_ref[:, 0, 0])
    den = sc + jnp.sum(p[:, :, 4, :]) + jnp.sum(t_ref[:, 0, 1])
    o_ref[...] = num / (den + _EPS)


def kernel(region_true, affinity_true, region_pred, affinity_pred,
           confidence, fg_mask, bg_mask):
    arrays = (region_true, affinity_true, region_pred, affinity_pred,
              confidence, fg_mask, bg_mask)
    sc_parts = _sc_partials(*arrays)
    tc_parts = _tc_partials(*arrays)
    out = pl.pallas_call(
        _combine_body,
        out_shape=jax.ShapeDtypeStruct((), jnp.float32),
        out_specs=pl.BlockSpec(memory_space=pltpu.SMEM),
    )(sc_parts, tc_parts)
    return out


# speedup vs baseline: 3.2686x; 1.0739x over previous
import functools

import jax
import jax.numpy as jnp
from jax import lax
from jax.experimental import pallas as pl
from jax.experimental.pallas import tpu as pltpu
from jax.experimental.pallas import tpu_sc as plsc

_EPS = 1e-07
_B, _H, _W = 16, 384, 384
_SC_SAMPLES = 4
_SUB_PER_SAMPLE = 8
_SPAN = _H // _SUB_PER_SAMPLE
_CHUNK_ROWS = 8
_N_CHUNKS = _SPAN // _CHUNK_ROWS
_LANES = 16
_VPR = _W // _LANES


def _sc_body(rt, af, rp, ap, cf, fg, bg, out,
             b0_rt, b0_af, b0_rp, b0_ap, b0_cf, b0_fg, b0_bg,
             b1_rt, b1_af, b1_rp, b1_ap, b1_cf, b1_fg, b1_bg,
             b_out, sem0, sem1):
    wid = lax.axis_index("s") * 2 + lax.axis_index("c")
    sample = wid // _SUB_PER_SAMPLE
    quarter = wid - sample * _SUB_PER_SAMPLE
    row_base = quarter * _SPAN

    slots = ((b0_rt, b0_af, b0_rp, b0_ap, b0_cf, b0_fg, b0_bg, sem0),
             (b1_rt, b1_af, b1_rp, b1_ap, b1_cf, b1_fg, b1_bg, sem1))
    hbms = (rt, af, rp, ap, cf, fg, bg)

    def issue(chunk, slot):
        r0 = row_base + chunk * _CHUNK_ROWS
        sem = slot[7]
        for h, b in zip(hbms, slot[:7]):
            pltpu.async_copy(h.at[sample, pl.ds(r0, _CHUNK_ROWS), :], b, sem)

    def drain(slot):
        sem = slot[7]
        for h, b in zip(hbms, slot[:7]):
            pltpu.make_async_copy(
                h.at[sample, pl.ds(row_base, _CHUNK_ROWS), :], b, sem).wait()

    def compute(slot, carry):
        b_rt, b_af, b_rp, b_ap, b_cf, b_fg, b_bg = slot[:7]

        def row_step(r, c2):
            M, SL, SC, SLFG, SCFG = c2
            for j in range(_VPR):
                sl = pl.ds(j * _LANES, _LANES)
                vcf = b_cf[r, sl]
                conf = jnp.where(vcf >= 0.5, vcf, 0.0)
                l = (jnp.abs(b_rt[r, sl] - b_rp[r, sl])
                     + jnp.abs(b_af[r, sl] - b_ap[r, sl])) * conf
                vfg = b_fg[r, sl]
                vbg = b_bg[r, sl]
                nl = l * vbg
                tie = nl >= M
                rst = nl > M
                M = jnp.maximum(M, nl)
                SLFG = SLFG + l * vfg
                SCFG = SCFG + conf * vfg
                SL = jnp.where(rst, 0.0, SL) + jnp.where(tie, nl, 0.0)
                SC = (jnp.where(rst, 0.0, SC)
                      + jnp.where(tie, conf * vbg, 0.0))
            return (M, SL, SC, SLFG, SCFG)

        return lax.fori_loop(0, _CHUNK_ROWS, row_step, carry)

    issue(0, slots[0])
    issue(1, slots[1])

    def pair_step(g, carry):
        for p in range(2):
            slot = slots[p]
            drain(slot)
            carry = compute(slot, carry)

            @pl.when(g < _N_CHUNKS // 2 - 1)
            def _():
                issue(2 * g + 2 + p, slot)
        return carry

    z = jnp.zeros((_LANES,), jnp.float32)
    M, SL, SC, SLFG, SCFG = lax.fori_loop(
        0, _N_CHUNKS // 2, pair_step, (z, z, z, z, z))

    b_out[0, :] = M
    b_out[1, :] = SL
    b_out[2, :] = SC
    b_out[3, :] = SLFG
    b_out[4, :] = SCFG
    pltpu.sync_copy(b_out, out.at[wid])


@functools.partial(
    pl.kernel,
    out_type=jax.ShapeDtypeStruct((32, 5, _LANES), jnp.float32),
    mesh=plsc.VectorSubcoreMesh(core_axis_name="c", subcore_axis_name="s"),
    scratch_types=(
        [pltpu.VMEM((_CHUNK_ROWS, _W), jnp.float32)] * 14
        + [pltpu.VMEM((5, _LANES), jnp.float32)]
        + [pltpu.SemaphoreType.DMA, pltpu.SemaphoreType.DMA]
    ),
)
def _sc_partials(*args):
    _sc_body(*args)


def _tc_part_body(rt, af, rp, ap, cf, fg, bg, o_ref):
    vcf = cf[...]
    conf = jnp.where(vcf >= 0.5, vcf, 0.0)
    l = (jnp.abs(rt[...] - rp[...]) + jnp.abs(af[...] - ap[...])) * conf
    vfg = fg[...]
    vbg = bg[...]
    nl = l * vbg
    m = jnp.max(nl)
    hard = (vbg > 0.0) & (nl >= m)
    num = jnp.sum(l * vfg) + jnp.sum(jnp.where(hard, l, 0.0))
    den = jnp.sum(conf * vfg) + jnp.sum(jnp.where(hard, conf, 0.0))
    col = lax.broadcasted_iota(jnp.int32, (1, 1, 128), 2)
    o_ref[...] = jnp.where(col == 0, num, jnp.where(col == 1, den, 0.0))


def _tc_partials(*arrays):
    spec = pl.BlockSpec((1, _H, _W), lambda i: (i + _SC_SAMPLES, 0, 0))
    return pl.pallas_call(
        _tc_part_body,
        grid=(_B - _SC_SAMPLES,),
        in_specs=[spec] * 7,
        out_specs=pl.BlockSpec((1, 1, 128), lambda i: (i, 0, 0)),
        out_shape=jax.ShapeDtypeStruct(
            (_B - _SC_SAMPLES, 1, 128), jnp.float32),
    )(*arrays)


def _combine_body(p_ref, t_ref, o_ref):
    p = p_ref[...].reshape(_SC_SAMPLES, _SUB_PER_SAMPLE, 5, _LANES)
    M = p[:, :, 0, :]
    m = jnp.max(M.reshape(_SC_SAMPLES, -1), axis=1)[:, None, None]
    w = M >= m
    sl = jnp.sum(jnp.where(w, p[:, :, 1, :], 0.0))
    sc = jnp.sum(jnp.where(w, p[:, :, 2, :], 0.0))
    num = sl + jnp.sum(p[:, :, 3, :]) + jnp.sum(t_ref[:, 0, 0])
    den = sc + jnp.sum(p[:, :, 4, :]) + jnp.sum(t_ref[:, 0, 1])
    o_ref[...] = num / (den + _EPS)


def kernel(region_true, affinity_true, region_pred, affinity_pred,
           confidence, fg_mask, bg_mask):
    arrays = (region_true, affinity_true, region_pred, affinity_pred,
              confidence, fg_mask, bg_mask)
    sc_parts = _sc_partials(*arrays)
    tc_parts = _tc_partials(*arrays)
    out = pl.pallas_call(
        _combine_body,
        out_shape=jax.ShapeDtypeStruct((), jnp.float32),
        out_specs=pl.BlockSpec(memory_space=pltpu.SMEM),
    )(sc_parts, tc_parts)
    return out


# speedup vs baseline: 3.3065x; 1.0116x over previous
import functools

import jax
import jax.numpy as jnp
from jax import lax
from jax.experimental import pallas as pl
from jax.experimental.pallas import tpu as pltpu
from jax.experimental.pallas import tpu_sc as plsc

_EPS = 1e-07
_B, _H, _W = 16, 384, 384
_SC_SAMPLES = 4
_SUB_PER_SAMPLE = 8
_SPAN = _H // _SUB_PER_SAMPLE
_CHUNK_ROWS = 8
_N_CHUNKS = _SPAN // _CHUNK_ROWS
_LANES = 16
_VPR = _W // _LANES


def _sc_body(rt, af, rp, ap, cf, fg, bg, out,
             b0_rt, b0_af, b0_rp, b0_ap, b0_cf, b0_fg, b0_bg,
             b1_rt, b1_af, b1_rp, b1_ap, b1_cf, b1_fg, b1_bg,
             b_out, sem0, sem1):
    wid = lax.axis_index("s") * 2 + lax.axis_index("c")
    sample = wid // _SUB_PER_SAMPLE
    quarter = wid - sample * _SUB_PER_SAMPLE
    row_base = quarter * _SPAN

    slots = ((b0_rt, b0_af, b0_rp, b0_ap, b0_cf, b0_fg, b0_bg, sem0),
             (b1_rt, b1_af, b1_rp, b1_ap, b1_cf, b1_fg, b1_bg, sem1))
    hbms = (rt, af, rp, ap, cf, fg, bg)

    def issue(chunk, slot):
        r0 = row_base + chunk * _CHUNK_ROWS
        sem = slot[7]
        for h, b in zip(hbms, slot[:7]):
            pltpu.async_copy(h.at[sample, pl.ds(r0, _CHUNK_ROWS), :], b, sem)

    def drain(slot):
        sem = slot[7]
        for h, b in zip(hbms, slot[:7]):
            pltpu.make_async_copy(
                h.at[sample, pl.ds(row_base, _CHUNK_ROWS), :], b, sem).wait()

    def compute(slot, carry):
        b_rt, b_af, b_rp, b_ap, b_cf, b_fg, b_bg = slot[:7]

        def row_step(r, c2):
            M, SL, SC, SLFG, SCFG = c2
            for j in range(_VPR):
                sl = pl.ds(j * _LANES, _LANES)
                vcf = b_cf[r, sl]
                conf = jnp.where(vcf >= 0.5, vcf, 0.0)
                l = (jnp.abs(b_rt[r, sl] - b_rp[r, sl])
                     + jnp.abs(b_af[r, sl] - b_ap[r, sl])) * conf
                vfg = b_fg[r, sl]
                vbg = b_bg[r, sl]
                nl = l * vbg
                tie = nl >= M
                rst = nl > M
                M = jnp.maximum(M, nl)
                SLFG = SLFG + l * vfg
                SCFG = SCFG + conf * vfg
                SL = jnp.where(rst, 0.0, SL) + jnp.where(tie, nl, 0.0)
                SC = (jnp.where(rst, 0.0, SC)
                      + jnp.where(tie, conf * vbg, 0.0))
            return (M, SL, SC, SLFG, SCFG)

        return lax.fori_loop(0, _CHUNK_ROWS, row_step, carry)

    issue(0, slots[0])
    issue(1, slots[1])

    def pair_step(g, carry):
        for p in range(2):
            slot = slots[p]
            drain(slot)
            carry = compute(slot, carry)

            @pl.when(g < _N_CHUNKS // 2 - 1)
            def _():
                issue(2 * g + 2 + p, slot)
        return carry

    z = jnp.zeros((_LANES,), jnp.float32)
    M, SL, SC, SLFG, SCFG = lax.fori_loop(
        0, _N_CHUNKS // 2, pair_step, (z, z, z, z, z))

    b_out[0, :] = M
    b_out[1, :] = SL
    b_out[2, :] = SC
    b_out[3, :] = SLFG
    b_out[4, :] = SCFG
    pltpu.sync_copy(b_out, out.at[wid])


@functools.partial(
    pl.kernel,
    out_type=jax.ShapeDtypeStruct((32, 5, _LANES), jnp.float32),
    mesh=plsc.VectorSubcoreMesh(core_axis_name="c", subcore_axis_name="s"),
    scratch_types=(
        [pltpu.VMEM((_CHUNK_ROWS, _W), jnp.float32)] * 14
        + [pltpu.VMEM((5, _LANES), jnp.float32)]
        + [pltpu.SemaphoreType.DMA, pltpu.SemaphoreType.DMA]
    ),
)
def _sc_partials(*args):
    _sc_body(*args)


def _tc_part_body(rt, af, rp, ap, cf, fg, bg, o_ref):
    vcf = cf[...]
    conf = jnp.where(vcf >= 0.5, vcf, 0.0)
    l = (jnp.abs(rt[...] - rp[...]) + jnp.abs(af[...] - ap[...])) * conf
    vfg = fg[...]
    vbg = bg[...]
    nl = l * vbg
    m = jnp.max(nl)
    hardf = jnp.where((vbg > 0.0) & (nl >= m), 1.0, 0.0)
    t = vfg + hardf
    num = jnp.sum(l * t)
    den = jnp.sum(conf * t)
    col = lax.broadcasted_iota(jnp.int32, (1, 1, 128), 2)
    o_ref[...] = jnp.where(col == 0, num, jnp.where(col == 1, den, 0.0))


def _tc_partials(*arrays):
    spec = pl.BlockSpec((1, _H, _W), lambda i: (i + _SC_SAMPLES, 0, 0))
    return pl.pallas_call(
        _tc_part_body,
        grid=(_B - _SC_SAMPLES,),
        in_specs=[spec] * 7,
        out_specs=pl.BlockSpec((1, 1, 128), lambda i: (i, 0, 0)),
        out_shape=jax.ShapeDtypeStruct(
            (_B - _SC_SAMPLES, 1, 128), jnp.float32),
    )(*arrays)


def _combine_body(p_ref, t_ref, o_ref):
    p = p_ref[...].reshape(_SC_SAMPLES, _SUB_PER_SAMPLE, 5, _LANES)
    M = p[:, :, 0, :]
    m = jnp.max(M.reshape(_SC_SAMPLES, -1), axis=1)[:, None, None]
    w = M >= m
    sl = jnp.sum(jnp.where(w, p[:, :, 1, :], 0.0))
    sc = jnp.sum(jnp.where(w, p[:, :, 2, :], 0.0))
    num = sl + jnp.sum(p[:, :, 3, :]) + jnp.sum(t_ref[:, 0, 0])
    den = sc + jnp.sum(p[:, :, 4, :]) + jnp.sum(t_ref[:, 0, 1])
    o_ref[...] = num / (den + _EPS)


def kernel(region_true, affinity_true, region_pred, affinity_pred,
           confidence, fg_mask, bg_mask):
    arrays = (region_true, affinity_true, region_pred, affinity_pred,
              confidence, fg_mask, bg_mask)
    sc_parts = _sc_partials(*arrays)
    tc_parts = _tc_partials(*arrays)
    out = pl.pallas_call(
        _combine_body,
        out_shape=jax.ShapeDtypeStruct((), jnp.float32),
        out_specs=pl.BlockSpec(memory_space=pltpu.SMEM),
    )(sc_parts, tc_parts)
    return out


# speedup vs baseline: 3.4329x; 1.0382x over previous
import functools

import jax
import jax.numpy as jnp
from jax import lax
from jax.experimental import pallas as pl
from jax.experimental.pallas import tpu as pltpu
from jax.experimental.pallas import tpu_sc as plsc

_EPS = 1e-07
_B, _H, _W = 16, 384, 384
_SC_SAMPLES = 4
_SUB_PER_SAMPLE = 8
_SPAN = _H // _SUB_PER_SAMPLE
_CHUNK_ROWS = 8
_N_CHUNKS = _SPAN // _CHUNK_ROWS
_LANES = 16
_VPR = _W // _LANES


def _sc_body(rt, af, rp, ap, cf, fg, bg, out,
             b0_rt, b0_af, b0_rp, b0_ap, b0_cf, b0_fg, b0_bg,
             b1_rt, b1_af, b1_rp, b1_ap, b1_cf, b1_fg, b1_bg,
             b_out, sem0, sem1):
    wid = lax.axis_index("s") * 2 + lax.axis_index("c")
    sample = wid // _SUB_PER_SAMPLE
    quarter = wid - sample * _SUB_PER_SAMPLE
    row_base = quarter * _SPAN

    slots = ((b0_rt, b0_af, b0_rp, b0_ap, b0_cf, b0_fg, b0_bg, sem0),
             (b1_rt, b1_af, b1_rp, b1_ap, b1_cf, b1_fg, b1_bg, sem1))
    hbms = (rt, af, rp, ap, cf, fg, bg)

    def issue(chunk, slot):
        r0 = row_base + chunk * _CHUNK_ROWS
        sem = slot[7]
        for h, b in zip(hbms, slot[:7]):
            pltpu.async_copy(h.at[sample, pl.ds(r0, _CHUNK_ROWS), :], b, sem)

    def drain(slot):
        sem = slot[7]
        for h, b in zip(hbms, slot[:7]):
            pltpu.make_async_copy(
                h.at[sample, pl.ds(row_base, _CHUNK_ROWS), :], b, sem).wait()

    def compute(slot, carry):
        b_rt, b_af, b_rp, b_ap, b_cf, b_fg, b_bg = slot[:7]

        def row_step(r, c2):
            M, SL, SC, SLFG, SCFG = c2
            for j in range(_VPR):
                sl = pl.ds(j * _LANES, _LANES)
                vcf = b_cf[r, sl]
                conf = jnp.where(vcf >= 0.5, vcf, 0.0)
                l = (jnp.abs(b_rt[r, sl] - b_rp[r, sl])
                     + jnp.abs(b_af[r, sl] - b_ap[r, sl])) * conf
                vfg = b_fg[r, sl]
                vbg = b_bg[r, sl]
                nl = l * vbg
                tie = nl >= M
                rst = nl > M
                M = jnp.maximum(M, nl)
                SLFG = SLFG + l * vfg
                SCFG = SCFG + conf * vfg
                SL = jnp.where(rst, 0.0, SL) + jnp.where(tie, nl, 0.0)
                SC = (jnp.where(rst, 0.0, SC)
                      + jnp.where(tie, conf * vbg, 0.0))
            return (M, SL, SC, SLFG, SCFG)

        return lax.fori_loop(0, _CHUNK_ROWS, row_step, carry)

    issue(0, slots[0])
    issue(1, slots[1])

    def pair_step(g, carry):
        for p in range(2):
            slot = slots[p]
            drain(slot)
            carry = compute(slot, carry)

            @pl.when(g < _N_CHUNKS // 2 - 1)
            def _():
                issue(2 * g + 2 + p, slot)
        return carry

    z = jnp.zeros((_LANES,), jnp.float32)
    M, SL, SC, SLFG, SCFG = lax.fori_loop(
        0, _N_CHUNKS // 2, pair_step, (z, z, z, z, z))

    b_out[0, :] = M
    b_out[1, :] = SL
    b_out[2, :] = SC
    b_out[3, :] = SLFG
    b_out[4, :] = SCFG
    pltpu.sync_copy(b_out, out.at[wid])


@functools.partial(
    pl.kernel,
    out_type=jax.ShapeDtypeStruct((32, 5, _LANES), jnp.float32),
    mesh=plsc.VectorSubcoreMesh(core_axis_name="c", subcore_axis_name="s"),
    scratch_types=(
        [pltpu.VMEM((_CHUNK_ROWS, _W), jnp.float32)] * 14
        + [pltpu.VMEM((5, _LANES), jnp.float32)]
        + [pltpu.SemaphoreType.DMA, pltpu.SemaphoreType.DMA]
    ),
)
def _sc_partials(*args):
    _sc_body(*args)


def _tc_part_body(rt, af, rp, ap, cf, fg, bg, o_ref):
    vcf = cf[...]
    conf = jnp.where(vcf >= 0.5, vcf, 0.0)
    l = (jnp.abs(rt[...] - rp[...]) + jnp.abs(af[...] - ap[...])) * conf
    vfg = fg[...]
    vbg = bg[...]
    nl = l * vbg
    m = jnp.max(nl, axis=(1, 2), keepdims=True)
    hardf = jnp.where((vbg > 0.0) & (nl >= m), 1.0, 0.0)
    t = vfg + hardf
    num = jnp.sum(l * t)
    den = jnp.sum(conf * t)
    col = lax.broadcasted_iota(jnp.int32, (1, 1, 128), 2)
    o_ref[...] = jnp.where(col == 0, num, jnp.where(col == 1, den, 0.0))


def _tc_partials(*arrays):
    spec = pl.BlockSpec((2, _H, _W), lambda i: (i + _SC_SAMPLES // 2, 0, 0))
    return pl.pallas_call(
        _tc_part_body,
        grid=((_B - _SC_SAMPLES) // 2,),
        in_specs=[spec] * 7,
        out_specs=pl.BlockSpec((1, 1, 128), lambda i: (i, 0, 0)),
        out_shape=jax.ShapeDtypeStruct(
            ((_B - _SC_SAMPLES) // 2, 1, 128), jnp.float32),
    )(*arrays)


def _combine_body(p_ref, t_ref, o_ref):
    p = p_ref[...].reshape(_SC_SAMPLES, _SUB_PER_SAMPLE, 5, _LANES)
    M = p[:, :, 0, :]
    m = jnp.max(M.reshape(_SC_SAMPLES, -1), axis=1)[:, None, None]
    w = M >= m
    sl = jnp.sum(jnp.where(w, p[:, :, 1, :], 0.0))
    sc = jnp.sum(jnp.where(w, p[:, :, 2, :], 0.0))
    num = sl + jnp.sum(p[:, :, 3, :]) + jnp.sum(t_ref[:, 0, 0])
    den = sc + jnp.sum(p[:, :, 4, :]) + jnp.sum(t_ref[:, 0, 1])
    o_ref[...] = num / (den + _EPS)


def kernel(region_true, affinity_true, region_pred, affinity_pred,
           confidence, fg_mask, bg_mask):
    arrays = (region_true, affinity_true, region_pred, affinity_pred,
              confidence, fg_mask, bg_mask)
    sc_parts = _sc_partials(*arrays)
    tc_parts = _tc_partials(*arrays)
    out = pl.pallas_call(
        _combine_body,
        out_shape=jax.ShapeDtypeStruct((), jnp.float32),
        out_specs=pl.BlockSpec(memory_space=pltpu.SMEM),
    )(sc_parts, tc_parts)
    return out
